# baseline (speedup 1.0000x reference)
import jax
import jax.numpy as jnp
from jax.experimental import pallas as pl
from jax.experimental.pallas import tpu as pltpu

_B, _N, _D = 16, 4096, 3
_G, _K = 256, 32
_BIG = 1e30
_CH = 128
_Q = 8


def _fps_body(x_ref, y_ref, z_ref, cidx_ref, cx_ref, cy_ref, cz_ref, dist_ref):
    x = x_ref[0]
    y = y_ref[0]
    z = z_ref[0]
    iota_n = jax.lax.broadcasted_iota(jnp.int32, (_B, _N), 1)
    iota_g = jax.lax.broadcasted_iota(jnp.int32, (_B, _G), 1)
    dist_ref[...] = jnp.full((_B, _N), 1e10, jnp.float32)
    cidx_ref[...] = jnp.zeros((_B, _G), jnp.int32)
    cx_ref[...] = jnp.zeros((_B, _G), jnp.float32)
    cy_ref[...] = jnp.zeros((_B, _G), jnp.float32)
    cz_ref[...] = jnp.zeros((_B, _G), jnp.float32)

    def body(i, carry):
        dist = dist_ref[...]
        m = jnp.max(dist, axis=1, keepdims=True)
        far = jnp.min(jnp.where(dist == m, iota_n, _N), axis=1, keepdims=True)
        oh_i = (iota_g == i).astype(jnp.int32)
        oh_f = oh_i.astype(jnp.float32)
        cidx_ref[...] = cidx_ref[...] + oh_i * far
        sel = iota_n == far
        fx = jnp.sum(jnp.where(sel, x, 0.0), axis=1, keepdims=True)
        fy = jnp.sum(jnp.where(sel, y, 0.0), axis=1, keepdims=True)
        fz = jnp.sum(jnp.where(sel, z, 0.0), axis=1, keepdims=True)
        cx_ref[...] = cx_ref[...] + oh_f * fx
        cy_ref[...] = cy_ref[...] + oh_f * fy
        cz_ref[...] = cz_ref[...] + oh_f * fz
        dx = x - fx
        dy = y - fy
        dz = z - fz
        d = (dx * dx + dy * dy) + dz * dz
        dist_ref[...] = jnp.minimum(dist, d)
        return carry

    jax.lax.fori_loop(0, _G, body, 0)


def _knn_body(x_ref, y_ref, z_ref, cx_ref, cy_ref, cz_ref, c3_ref, p3t_ref,
              idx_ref, d2_ref, d28_ref):
    x = x_ref[0]
    y = y_ref[0]
    z = z_ref[0]
    cx = cx_ref[0]
    cy = cy_ref[0]
    cz = cz_ref[0]
    psq = (x * x + y * y) + z * z
    csq = (cx * cx + cy * cy) + cz * cz
    dot = jax.lax.dot_general(
        c3_ref[0], p3t_ref[0], (((1,), (0,)), ((), ())),
        precision=jax.lax.Precision.DEFAULT,
        preferred_element_type=jnp.float32)
    d2_ref[...] = (csq + psq) - 2.0 * dot

    _NC = _N // _CH
    for c in range(_NC):
        d28_ref[pl.ds(c * _G, _G), :] = d2_ref[:, c * _CH:(c + 1) * _CH]

    iota_l = jax.lax.broadcasted_iota(jnp.int32, (_NC * _G, _CH), 1)
    row_c = jax.lax.broadcasted_iota(jnp.int32, (_NC * _G, 1), 0) // _G
    iota_q = jax.lax.broadcasted_iota(jnp.int32, (_NC * _G, _Q), 1)
    cv = jnp.full((_NC * _G, _Q), _BIG, jnp.float32)
    gv = jnp.zeros((_NC * _G, _Q), jnp.int32)
    for q in range(_Q):
        v = d28_ref[...]
        m = jnp.min(v, axis=1, keepdims=True)
        lane = jnp.min(jnp.where(v == m, iota_l, _CH), axis=1, keepdims=True)
        gidx = row_c * _CH + lane
        ohq = iota_q == q
        cv = jnp.where(ohq, jnp.broadcast_to(m, (_NC * _G, _Q)), cv)
        gv = jnp.where(ohq, jnp.broadcast_to(gidx, (_NC * _G, _Q)), gv)
        d28_ref[...] = jnp.where(iota_l == lane, _BIG, v)

    cv3 = cv.reshape(_NC, _G, _Q)
    gv3 = gv.reshape(_NC, _G, _Q)
    iota_c3 = jax.lax.broadcasted_iota(jnp.int32, (_NC, _G, _Q), 0)
    iota_q3 = jax.lax.broadcasted_iota(jnp.int32, (_NC, _G, _Q), 2)
    iota_q2 = jax.lax.broadcasted_iota(jnp.int32, (_G, _Q), 1)
    viol = jnp.zeros((1, 1), jnp.int32)
    for k in range(_K):
        mv = jnp.min(cv3, axis=0)
        cstar = jnp.min(jnp.where(cv3 == mv[None], iota_c3, _NC), axis=0)
        m = jnp.min(mv, axis=1, keepdims=True)
        is_m = mv == m
        cand_c = jnp.where(is_m, cstar, _NC)
        cmin = jnp.min(cand_c, axis=1, keepdims=True)
        qstar = jnp.min(
            jnp.where(is_m & (cstar == cmin), iota_q2, _Q), axis=1,
            keepdims=True)
        hit = (iota_c3 == cmin[None]) & (iota_q3 == qstar[None])
        g = jnp.sum(jnp.sum(jnp.where(hit, gv3, 0), axis=0), axis=1,
                    keepdims=True)
        idx_ref[0, :, pl.ds(k, 1)] = g
        cv3 = jnp.where(hit, _BIG, cv3)
        viol = viol + jnp.sum((qstar == _Q - 1).astype(jnp.int32),
                              axis=0, keepdims=True)
    viol = jnp.max(viol)

    @pl.when(viol >= _Q)
    def _fallback():
        d2f = (csq + psq) - 2.0 * dot
        iota_n = jax.lax.broadcasted_iota(jnp.int32, (_G, _N), 1)
        dcur = d2f
        for k in range(_K):
            m = jnp.min(dcur, axis=1, keepdims=True)
            sel = jnp.min(jnp.where(dcur == m, iota_n, _N), axis=1,
                          keepdims=True)
            idx_ref[0, :, pl.ds(k, 1)] = sel
            dcur = jnp.where(iota_n == sel, _BIG, dcur)


def kernel(xyz):
    xt = jnp.transpose(xyz, (2, 0, 1))
    x3 = xt[:, None]
    x = x3[0]
    y = x3[1]
    z = x3[2]

    fps = pl.pallas_call(
        _fps_body,
        grid=(1,),
        in_specs=[pl.BlockSpec((1, _B, _N), lambda i: (0, 0, 0))] * 3,
        out_specs=[pl.BlockSpec((_B, _G), lambda i: (0, 0))] * 4,
        out_shape=[
            jax.ShapeDtypeStruct((_B, _G), jnp.int32),
            jax.ShapeDtypeStruct((_B, _G), jnp.float32),
            jax.ShapeDtypeStruct((_B, _G), jnp.float32),
            jax.ShapeDtypeStruct((_B, _G), jnp.float32),
        ],
        scratch_shapes=[pltpu.VMEM((_B, _N), jnp.float32)],
    )
    cidx, cx, cy, cz = fps(x, y, z)

    knn = pl.pallas_call(
        _knn_body,
        grid=(_B,),
        in_specs=[
            pl.BlockSpec((1, 1, _N), lambda i: (i, 0, 0)),
            pl.BlockSpec((1, 1, _N), lambda i: (i, 0, 0)),
            pl.BlockSpec((1, 1, _N), lambda i: (i, 0, 0)),
            pl.BlockSpec((1, _G, 1), lambda i: (i, 0, 0)),
            pl.BlockSpec((1, _G, 1), lambda i: (i, 0, 0)),
            pl.BlockSpec((1, _G, 1), lambda i: (i, 0, 0)),
            pl.BlockSpec((1, _G, _D), lambda i: (i, 0, 0)),
            pl.BlockSpec((1, _D, _N), lambda i: (i, 0, 0)),
        ],
        out_specs=pl.BlockSpec((1, _G, _K), lambda i: (i, 0, 0)),
        out_shape=jax.ShapeDtypeStruct((_B, _G, _K), jnp.int32),
        scratch_shapes=[
            pltpu.VMEM((_G, _N), jnp.float32),
            pltpu.VMEM(((_N // _CH) * _G, _CH), jnp.float32),
        ],
    )
    center = jnp.stack([cx, cy, cz], axis=-1)
    p3t = jnp.transpose(xyz, (0, 2, 1))
    idx = knn(
        x.reshape(_B, 1, _N), y.reshape(_B, 1, _N), z.reshape(_B, 1, _N),
        cx[:, :, None], cy[:, :, None], cz[:, :, None],
        center, p3t,
    )
    return (idx, cidx, center)


# speedup vs baseline: 2.7577x; 2.7577x over previous
import jax
import jax.numpy as jnp
from jax.experimental import pallas as pl
from jax.experimental.pallas import tpu as pltpu

_B, _N, _D = 16, 4096, 3
_G, _K = 256, 32
_BIG = 1e30
_CH = 128
_Q = 8


def _fps_body(x_ref, y_ref, z_ref, cidx_ref, cx_ref, cy_ref, cz_ref, dist_ref):
    x = x_ref[0]
    y = y_ref[0]
    z = z_ref[0]
    iota_n = jax.lax.broadcasted_iota(jnp.int32, (_B, _N), 1)
    iota_g = jax.lax.broadcasted_iota(jnp.int32, (_B, _G), 1)
    dist_ref[...] = jnp.full((_B, _N), 1e10, jnp.float32)
    cidx_ref[...] = jnp.zeros((_B, _G), jnp.int32)
    cx_ref[...] = jnp.zeros((_B, _G), jnp.float32)
    cy_ref[...] = jnp.zeros((_B, _G), jnp.float32)
    cz_ref[...] = jnp.zeros((_B, _G), jnp.float32)

    def body(i, carry):
        dist = dist_ref[...]
        m = jnp.max(dist, axis=1, keepdims=True)
        far = jnp.min(jnp.where(dist == m, iota_n, _N), axis=1, keepdims=True)
        oh_i = (iota_g == i).astype(jnp.int32)
        oh_f = oh_i.astype(jnp.float32)
        cidx_ref[...] = cidx_ref[...] + oh_i * far
        sel = iota_n == far
        fx = jnp.sum(jnp.where(sel, x, 0.0), axis=1, keepdims=True)
        fy = jnp.sum(jnp.where(sel, y, 0.0), axis=1, keepdims=True)
        fz = jnp.sum(jnp.where(sel, z, 0.0), axis=1, keepdims=True)
        cx_ref[...] = cx_ref[...] + oh_f * fx
        cy_ref[...] = cy_ref[...] + oh_f * fy
        cz_ref[...] = cz_ref[...] + oh_f * fz
        dx = x - fx
        dy = y - fy
        dz = z - fz
        d = (dx * dx + dy * dy) + dz * dz
        dist_ref[...] = jnp.minimum(dist, d)
        return carry

    jax.lax.fori_loop(0, _G, body, 0)


def _knn_body(x_ref, y_ref, z_ref, cx_ref, cy_ref, cz_ref, c3t_ref, p3_ref,
              idxT_ref, d2t_ref, cvT_ref, gvT_ref):
    x = x_ref[0]
    y = y_ref[0]
    z = z_ref[0]
    cx = cx_ref[0]
    cy = cy_ref[0]
    cz = cz_ref[0]
    psq = (x * x + y * y) + z * z
    csq = (cx * cx + cy * cy) + cz * cz
    dot = jax.lax.dot_general(
        p3_ref[0], c3t_ref[0], (((1,), (0,)), ((), ())),
        precision=jax.lax.Precision.DEFAULT,
        preferred_element_type=jnp.float32)
    d2t_ref[...] = (csq + psq) - 2.0 * dot

    _NC = _N // _CH
    iota_s = jax.lax.broadcasted_iota(jnp.int32, (_CH, _G), 0)
    for c in range(_NC):
        v = d2t_ref[pl.ds(c * _CH, _CH), :]
        for q in range(_Q):
            m = jnp.min(v, axis=0, keepdims=True)
            srow = jnp.min(jnp.where(v == m, iota_s, _CH), axis=0,
                           keepdims=True)
            cvT_ref[pl.ds(c * _Q + q, 1), :] = m
            gvT_ref[pl.ds(c * _Q + q, 1), :] = srow + c * _CH
            v = jnp.where(iota_s == srow, _BIG, v)

    _CC = _NC * _Q
    iota_r = jax.lax.broadcasted_iota(jnp.int32, (_CC, _G), 0)
    cv = cvT_ref[...]
    gv = gvT_ref[...]
    viol = jnp.zeros((1, _G), jnp.int32)
    for k in range(_K):
        m = jnp.min(cv, axis=0, keepdims=True)
        rstar = jnp.min(jnp.where(cv == m, iota_r, _CC), axis=0,
                        keepdims=True)
        g = jnp.sum(jnp.where(iota_r == rstar, gv, 0), axis=0, keepdims=True)
        idxT_ref[0, pl.ds(k, 1), :] = g
        cv = jnp.where(iota_r == rstar, _BIG, cv)
        viol = viol + (rstar % _Q == _Q - 1).astype(jnp.int32)
    nviol = jnp.max(viol)

    @pl.when(nviol > 0)
    def _fallback():
        iota_n = jax.lax.broadcasted_iota(jnp.int32, (_N, _G), 0)
        dcur = d2t_ref[...]
        for k in range(_K):
            m = jnp.min(dcur, axis=0, keepdims=True)
            sel = jnp.min(jnp.where(dcur == m, iota_n, _N), axis=0,
                          keepdims=True)
            idxT_ref[0, pl.ds(k, 1), :] = sel
            dcur = jnp.where(iota_n == sel, _BIG, dcur)


def kernel(xyz):
    xt = jnp.transpose(xyz, (2, 0, 1))
    x3 = xt[:, None]
    x = x3[0]
    y = x3[1]
    z = x3[2]

    fps = pl.pallas_call(
        _fps_body,
        grid=(1,),
        in_specs=[pl.BlockSpec((1, _B, _N), lambda i: (0, 0, 0))] * 3,
        out_specs=[pl.BlockSpec((_B, _G), lambda i: (0, 0))] * 4,
        out_shape=[
            jax.ShapeDtypeStruct((_B, _G), jnp.int32),
            jax.ShapeDtypeStruct((_B, _G), jnp.float32),
            jax.ShapeDtypeStruct((_B, _G), jnp.float32),
            jax.ShapeDtypeStruct((_B, _G), jnp.float32),
        ],
        scratch_shapes=[pltpu.VMEM((_B, _N), jnp.float32)],
    )
    cidx, cx, cy, cz = fps(x, y, z)

    knn = pl.pallas_call(
        _knn_body,
        grid=(_B,),
        in_specs=[
            pl.BlockSpec((1, _N, 1), lambda i: (i, 0, 0)),
            pl.BlockSpec((1, _N, 1), lambda i: (i, 0, 0)),
            pl.BlockSpec((1, _N, 1), lambda i: (i, 0, 0)),
            pl.BlockSpec((1, 1, _G), lambda i: (i, 0, 0)),
            pl.BlockSpec((1, 1, _G), lambda i: (i, 0, 0)),
            pl.BlockSpec((1, 1, _G), lambda i: (i, 0, 0)),
            pl.BlockSpec((1, _D, _G), lambda i: (i, 0, 0)),
            pl.BlockSpec((1, _N, _D), lambda i: (i, 0, 0)),
        ],
        out_specs=pl.BlockSpec((1, _K, _G), lambda i: (i, 0, 0)),
        out_shape=jax.ShapeDtypeStruct((_B, _K, _G), jnp.int32),
        scratch_shapes=[
            pltpu.VMEM((_N, _G), jnp.float32),
            pltpu.VMEM(((_N // _CH) * _Q, _G), jnp.float32),
            pltpu.VMEM(((_N // _CH) * _Q, _G), jnp.int32),
        ],
    )
    center = jnp.stack([cx, cy, cz], axis=-1)
    c3t = jnp.stack([cx, cy, cz], axis=1)
    idxT = knn(
        x.reshape(_B, _N, 1), y.reshape(_B, _N, 1), z.reshape(_B, _N, 1),
        cx[:, None, :], cy[:, None, :], cz[:, None, :],
        c3t, xyz,
    )
    idx = jnp.transpose(idxT, (0, 2, 1))
    return (idx, cidx, center)
